# baseline (speedup 1.0000x reference)
import functools

import jax
import jax.numpy as jnp
from jax import lax
from jax.experimental import pallas as pl
from jax.experimental.pallas import tpu as pltpu

N_BINS = 15
N_STREAMS = 4
BR = 512
_EPS = float(jnp.finfo(jnp.float32).eps)
_NEG = -3.0e38


def _one_block(x_ref, lab_ref, ones_ref, cnt_ref, acc_ref, conf_ref):
    x = x_ref[...]
    br, c = x.shape
    m = jnp.max(x, axis=1, keepdims=True)
    d = x - m
    iota = lax.broadcasted_iota(jnp.int32, (br, c), 1)
    lm = iota == lab_ref[...]
    e = jnp.exp(d)
    dl = jnp.where(lm, d, 0.0)
    s = lax.dot_general(
        e, ones_ref[...], (((1,), (0,)), ((), ())),
        preferred_element_type=jnp.float32)[:, 0]
    d_lab = lax.dot_general(
        dl, ones_ref[...], (((1,), (0,)), ((), ())),
        preferred_element_type=jnp.float32)[:, 0]
    matched = (d_lab >= 0.0).astype(jnp.float32)
    conf = jnp.clip(1.0 / s, _EPS, 1.0 - _EPS)
    bin_idx = jnp.clip(jnp.floor(conf * N_BINS).astype(jnp.int32), 0, N_BINS - 1)
    lanes = lax.broadcasted_iota(jnp.int32, (br, 16), 1)
    onehot = (bin_idx[:, None] == lanes).astype(jnp.float32)
    cnt_ref[...] += jnp.sum(onehot, axis=0, keepdims=True)
    acc_ref[...] += jnp.sum(onehot * matched[:, None], axis=0, keepdims=True)
    conf_ref[...] += jnp.sum(onehot * conf[:, None], axis=0, keepdims=True)


def _tc_body(num_blocks, *refs):
    x_refs = refs[:N_STREAMS]
    lab_refs = refs[N_STREAMS:2 * N_STREAMS]
    out_ref = refs[2 * N_STREAMS]
    cnt_ref, acc_ref, conf_ref, ones_ref = refs[2 * N_STREAMS + 1:]
    i = pl.program_id(0)

    @pl.when(i == 0)
    def _init():
        cnt_ref[...] = jnp.zeros_like(cnt_ref)
        acc_ref[...] = jnp.zeros_like(acc_ref)
        conf_ref[...] = jnp.zeros_like(conf_ref)
        ones_ref[...] = jnp.ones_like(ones_ref)

    for k in range(N_STREAMS):
        _one_block(x_refs[k], lab_refs[k], ones_ref, cnt_ref, acc_ref, conf_ref)

    @pl.when(i == num_blocks - 1)
    def _finish():
        out_ref[0, :] = cnt_ref[0, :]
        out_ref[1, :] = acc_ref[0, :]
        out_ref[2, :] = conf_ref[0, :]


def _combine_body(tc_ref, out_ref):
    counts = tc_ref[0, :]
    sum_acc = tc_ref[1, :]
    sum_conf = tc_ref[2, :]
    valid = counts >= 1.0
    safe = jnp.maximum(counts, 1.0)
    acc_h = jnp.where(valid, sum_acc / safe, 0.0)
    conf_h = jnp.where(valid, sum_conf / safe, 0.0)
    dev = jnp.sum(jnp.abs(acc_h - conf_h))
    non_empty = jnp.sum((counts != 0.0).astype(jnp.float32))
    bin_map = jnp.where(non_empty != 0.0,
                        dev / jnp.where(non_empty != 0.0, non_empty, 1.0),
                        0.0)
    total = jnp.sum(counts)
    denom = (total != 0.0).astype(jnp.float32)
    out_ref[0, 0] = jnp.where(denom != 0.0, bin_map / jnp.maximum(denom, 1.0),
                              0.0)


def kernel(logits, labels):
    n, c = logits.shape
    labels32 = labels.astype(jnp.int32)

    num_blocks = n // BR // N_STREAMS
    labels2d = labels32.reshape(n, 1)

    def xmap(k):
        return lambda i: (i + k * num_blocks, 0)

    tc_hist = pl.pallas_call(
        functools.partial(_tc_body, num_blocks),
        grid=(num_blocks,),
        in_specs=[pl.BlockSpec((BR, c), xmap(k)) for k in range(N_STREAMS)]
        + [pl.BlockSpec((BR, 1), xmap(k)) for k in range(N_STREAMS)],
        out_specs=pl.BlockSpec((3, 16), lambda i: (0, 0)),
        out_shape=jax.ShapeDtypeStruct((3, 16), jnp.float32),
        scratch_shapes=[pltpu.VMEM((1, 16), jnp.float32)] * 3
        + [pltpu.VMEM((c, 128), jnp.float32)],
    )(*([logits] * N_STREAMS + [labels2d] * N_STREAMS))

    out = pl.pallas_call(
        _combine_body,
        out_specs=pl.BlockSpec(memory_space=pltpu.SMEM),
        out_shape=jax.ShapeDtypeStruct((1, 1), jnp.float32),
    )(tc_hist)
    return out[0, 0]


# speedup vs baseline: 1.0014x; 1.0014x over previous
import functools

import jax
import jax.numpy as jnp
from jax import lax
from jax.experimental import pallas as pl
from jax.experimental.pallas import tpu as pltpu

N_BINS = 15
N_STREAMS = 8
BR = 512
_EPS = float(jnp.finfo(jnp.float32).eps)
_NEG = -3.0e38


def _one_block(x_ref, lab_ref, ones_ref, cnt_ref, acc_ref, conf_ref):
    x = x_ref[...]
    br, c = x.shape
    m = jnp.max(x, axis=1, keepdims=True)
    d = x - m
    iota = lax.broadcasted_iota(jnp.int32, (br, c), 1)
    lm = iota == lab_ref[...]
    e = jnp.exp(d)
    dl = jnp.where(lm, d, 0.0)
    s = lax.dot_general(
        e, ones_ref[...], (((1,), (0,)), ((), ())),
        preferred_element_type=jnp.float32)[:, 0]
    d_lab = lax.dot_general(
        dl, ones_ref[...], (((1,), (0,)), ((), ())),
        preferred_element_type=jnp.float32)[:, 0]
    matched = (d_lab >= 0.0).astype(jnp.float32)
    conf = jnp.clip(1.0 / s, _EPS, 1.0 - _EPS)
    bin_idx = jnp.clip(jnp.floor(conf * N_BINS).astype(jnp.int32), 0, N_BINS - 1)
    lanes = lax.broadcasted_iota(jnp.int32, (br, 16), 1)
    onehot = (bin_idx[:, None] == lanes).astype(jnp.float32)
    cnt_ref[...] += jnp.sum(onehot, axis=0, keepdims=True)
    acc_ref[...] += jnp.sum(onehot * matched[:, None], axis=0, keepdims=True)
    conf_ref[...] += jnp.sum(onehot * conf[:, None], axis=0, keepdims=True)


def _tc_body(num_blocks, *refs):
    x_refs = refs[:N_STREAMS]
    lab_refs = refs[N_STREAMS:2 * N_STREAMS]
    out_ref = refs[2 * N_STREAMS]
    cnt_ref, acc_ref, conf_ref, ones_ref = refs[2 * N_STREAMS + 1:]
    i = pl.program_id(0)

    @pl.when(i == 0)
    def _init():
        cnt_ref[...] = jnp.zeros_like(cnt_ref)
        acc_ref[...] = jnp.zeros_like(acc_ref)
        conf_ref[...] = jnp.zeros_like(conf_ref)
        ones_ref[...] = jnp.ones_like(ones_ref)

    for k in range(N_STREAMS):
        _one_block(x_refs[k], lab_refs[k], ones_ref, cnt_ref, acc_ref, conf_ref)

    @pl.when(i == num_blocks - 1)
    def _finish():
        out_ref[0, :] = cnt_ref[0, :]
        out_ref[1, :] = acc_ref[0, :]
        out_ref[2, :] = conf_ref[0, :]


def _combine_body(tc_ref, out_ref):
    counts = tc_ref[0, :]
    sum_acc = tc_ref[1, :]
    sum_conf = tc_ref[2, :]
    valid = counts >= 1.0
    safe = jnp.maximum(counts, 1.0)
    acc_h = jnp.where(valid, sum_acc / safe, 0.0)
    conf_h = jnp.where(valid, sum_conf / safe, 0.0)
    dev = jnp.sum(jnp.abs(acc_h - conf_h))
    non_empty = jnp.sum((counts != 0.0).astype(jnp.float32))
    bin_map = jnp.where(non_empty != 0.0,
                        dev / jnp.where(non_empty != 0.0, non_empty, 1.0),
                        0.0)
    total = jnp.sum(counts)
    denom = (total != 0.0).astype(jnp.float32)
    out_ref[0, 0] = jnp.where(denom != 0.0, bin_map / jnp.maximum(denom, 1.0),
                              0.0)


def kernel(logits, labels):
    n, c = logits.shape
    labels32 = labels.astype(jnp.int32)

    num_blocks = n // BR // N_STREAMS
    labels2d = labels32.reshape(n, 1)

    def xmap(k):
        return lambda i: (i + k * num_blocks, 0)

    tc_hist = pl.pallas_call(
        functools.partial(_tc_body, num_blocks),
        grid=(num_blocks,),
        in_specs=[pl.BlockSpec((BR, c), xmap(k)) for k in range(N_STREAMS)]
        + [pl.BlockSpec((BR, 1), xmap(k)) for k in range(N_STREAMS)],
        out_specs=pl.BlockSpec((3, 16), lambda i: (0, 0)),
        out_shape=jax.ShapeDtypeStruct((3, 16), jnp.float32),
        scratch_shapes=[pltpu.VMEM((1, 16), jnp.float32)] * 3
        + [pltpu.VMEM((c, 128), jnp.float32)],
    )(*([logits] * N_STREAMS + [labels2d] * N_STREAMS))

    out = pl.pallas_call(
        _combine_body,
        out_specs=pl.BlockSpec(memory_space=pltpu.SMEM),
        out_shape=jax.ShapeDtypeStruct((1, 1), jnp.float32),
    )(tc_hist)
    return out[0, 0]


# speedup vs baseline: 1.0069x; 1.0055x over previous
import functools

import jax
import jax.numpy as jnp
from jax import lax
from jax.experimental import pallas as pl
from jax.experimental.pallas import tpu as pltpu

N_BINS = 15
N_STREAMS = 4
BR = 1024
_EPS = float(jnp.finfo(jnp.float32).eps)
_NEG = -3.0e38


def _one_block(x_ref, lab_ref, ones_ref, cnt_ref, acc_ref, conf_ref):
    x = x_ref[...]
    br, c = x.shape
    m = jnp.max(x, axis=1, keepdims=True)
    d = x - m
    iota = lax.broadcasted_iota(jnp.int32, (br, c), 1)
    lm = iota == lab_ref[...]
    e = jnp.exp(d)
    dl = jnp.where(lm, d, 0.0)
    s = lax.dot_general(
        e, ones_ref[...], (((1,), (0,)), ((), ())),
        preferred_element_type=jnp.float32)[:, 0]
    d_lab = lax.dot_general(
        dl, ones_ref[...], (((1,), (0,)), ((), ())),
        preferred_element_type=jnp.float32)[:, 0]
    matched = (d_lab >= 0.0).astype(jnp.float32)
    conf = jnp.clip(1.0 / s, _EPS, 1.0 - _EPS)
    bin_idx = jnp.clip(jnp.floor(conf * N_BINS).astype(jnp.int32), 0, N_BINS - 1)
    lanes = lax.broadcasted_iota(jnp.int32, (br, 16), 1)
    onehot = (bin_idx[:, None] == lanes).astype(jnp.float32)
    cnt_ref[...] += jnp.sum(onehot, axis=0, keepdims=True)
    acc_ref[...] += jnp.sum(onehot * matched[:, None], axis=0, keepdims=True)
    conf_ref[...] += jnp.sum(onehot * conf[:, None], axis=0, keepdims=True)


def _tc_body(num_blocks, *refs):
    x_refs = refs[:N_STREAMS]
    lab_refs = refs[N_STREAMS:2 * N_STREAMS]
    out_ref = refs[2 * N_STREAMS]
    cnt_ref, acc_ref, conf_ref, ones_ref = refs[2 * N_STREAMS + 1:]
    i = pl.program_id(0)

    @pl.when(i == 0)
    def _init():
        cnt_ref[...] = jnp.zeros_like(cnt_ref)
        acc_ref[...] = jnp.zeros_like(acc_ref)
        conf_ref[...] = jnp.zeros_like(conf_ref)
        ones_ref[...] = jnp.ones_like(ones_ref)

    for k in range(N_STREAMS):
        _one_block(x_refs[k], lab_refs[k], ones_ref, cnt_ref, acc_ref, conf_ref)

    @pl.when(i == num_blocks - 1)
    def _finish():
        out_ref[0, :] = cnt_ref[0, :]
        out_ref[1, :] = acc_ref[0, :]
        out_ref[2, :] = conf_ref[0, :]


def _combine_body(tc_ref, out_ref):
    counts = tc_ref[0, :]
    sum_acc = tc_ref[1, :]
    sum_conf = tc_ref[2, :]
    valid = counts >= 1.0
    safe = jnp.maximum(counts, 1.0)
    acc_h = jnp.where(valid, sum_acc / safe, 0.0)
    conf_h = jnp.where(valid, sum_conf / safe, 0.0)
    dev = jnp.sum(jnp.abs(acc_h - conf_h))
    non_empty = jnp.sum((counts != 0.0).astype(jnp.float32))
    bin_map = jnp.where(non_empty != 0.0,
                        dev / jnp.where(non_empty != 0.0, non_empty, 1.0),
                        0.0)
    total = jnp.sum(counts)
    denom = (total != 0.0).astype(jnp.float32)
    out_ref[0, 0] = jnp.where(denom != 0.0, bin_map / jnp.maximum(denom, 1.0),
                              0.0)


def kernel(logits, labels):
    n, c = logits.shape
    labels32 = labels.astype(jnp.int32)

    num_blocks = n // BR // N_STREAMS
    labels2d = labels32.reshape(n, 1)

    def xmap(k):
        return lambda i: (i + k * num_blocks, 0)

    tc_hist = pl.pallas_call(
        functools.partial(_tc_body, num_blocks),
        grid=(num_blocks,),
        in_specs=[pl.BlockSpec((BR, c), xmap(k)) for k in range(N_STREAMS)]
        + [pl.BlockSpec((BR, 1), xmap(k)) for k in range(N_STREAMS)],
        out_specs=pl.BlockSpec((3, 16), lambda i: (0, 0)),
        out_shape=jax.ShapeDtypeStruct((3, 16), jnp.float32),
        scratch_shapes=[pltpu.VMEM((1, 16), jnp.float32)] * 3
        + [pltpu.VMEM((c, 128), jnp.float32)],
    )(*([logits] * N_STREAMS + [labels2d] * N_STREAMS))

    out = pl.pallas_call(
        _combine_body,
        out_specs=pl.BlockSpec(memory_space=pltpu.SMEM),
        out_shape=jax.ShapeDtypeStruct((1, 1), jnp.float32),
    )(tc_hist)
    return out[0, 0]
